# baseline (speedup 1.0000x reference)
import functools

import jax
import jax.numpy as jnp
from jax import lax
from jax.experimental import pallas as pl
from jax.experimental.pallas import tpu as pltpu
from jax.experimental.pallas import tpu_sc as plsc

TILE_M = 4096
TILE_N = 512


def _main_body(z_ref, cbt_ref, nz_ref, ne_ref, sim_ref, ids_ref, loss_ref,
               runmin_ref, runids_ref):
    j = pl.program_id(1)
    nj = pl.num_programs(1)
    z = z_ref[...]
    cbt = cbt_ref[...]
    dot = lax.dot_general(z, cbt, (((1,), (0,)), ((), ())),
                          preferred_element_type=jnp.float32)
    nz = nz_ref[...]
    ne = ne_ref[...]
    sim_ref[...] = dot * lax.rsqrt(nz) * lax.rsqrt(ne)
    dist = (-2.0 * dot + nz) + ne
    lmin = jnp.min(dist, axis=1, keepdims=True)
    col = lax.broadcasted_iota(jnp.int32, dist.shape, 1)
    larg = jnp.min(jnp.where(dist == lmin, col, jnp.int32(2 ** 30)),
                   axis=1, keepdims=True) + j * TILE_N

    @pl.when(j == 0)
    def _():
        runmin_ref[...] = lmin
        runids_ref[...] = larg

    @pl.when(j > 0)
    def _():
        better = lmin < runmin_ref[...]
        runmin_ref[...] = jnp.where(better, lmin, runmin_ref[...])
        runids_ref[...] = jnp.where(better, larg, runids_ref[...])

    @pl.when(j == nj - 1)
    def _():
        ids_ref[...] = runids_ref[...].T.reshape(1, 1, TILE_M)
        loss_ref[0, 0, 0] = jnp.sum(jnp.sqrt(jnp.maximum(runmin_ref[...], 0.0)))


def _main_call(z2d, cbt_in, nz_in, ne_in, interpret=False):
    m, c = z2d.shape
    k = cbt_in.shape[1]
    gm = m // TILE_M
    grid = (gm, k // TILE_N)
    return pl.pallas_call(
        _main_body,
        grid=grid,
        in_specs=[
            pl.BlockSpec((TILE_M, c), lambda i, j: (i, 0)),
            pl.BlockSpec((c, TILE_N), lambda i, j: (0, j)),
            pl.BlockSpec((TILE_M, 1), lambda i, j: (i, 0)),
            pl.BlockSpec((1, TILE_N), lambda i, j: (0, j)),
        ],
        out_specs=[
            pl.BlockSpec((TILE_M, TILE_N), lambda i, j: (i, j)),
            pl.BlockSpec((1, 1, TILE_M), lambda i, j: (i, 0, 0)),
            pl.BlockSpec((1, 1, 1), lambda i, j: (i, 0, 0),
                         memory_space=pltpu.SMEM),
        ],
        out_shape=[
            jax.ShapeDtypeStruct((m, k), jnp.float32),
            jax.ShapeDtypeStruct((gm, 1, TILE_M), jnp.int32),
            jax.ShapeDtypeStruct((gm, 1, 1), jnp.float32),
        ],
        scratch_shapes=[pltpu.VMEM((TILE_M, 1), jnp.float32),
                        pltpu.VMEM((TILE_M, 1), jnp.int32)],
        compiler_params=pltpu.CompilerParams(
            dimension_semantics=("parallel", "arbitrary")),
        interpret=interpret,
    )(z2d, cbt_in, nz_in, ne_in)


def _gather_call(codebook, ids):
    info = plsc.get_sparse_core_info()
    nc, ns = info.num_cores, info.num_subcores
    nw = nc * ns
    m = ids.shape[0]
    d = codebook.shape[1]
    b_per_w = m // nw
    chunk = 128
    nchunks = b_per_w // chunk
    mesh = plsc.VectorSubcoreMesh(core_axis_name="c", subcore_axis_name="s")

    @functools.partial(
        pl.kernel,
        mesh=mesh,
        out_type=jax.ShapeDtypeStruct((m, d), jnp.float32),
        scratch_types=[
            pltpu.VMEM((chunk,), jnp.int32),
            pltpu.VMEM((chunk, d), jnp.float32),
            pltpu.SemaphoreType.DMA,
        ],
    )
    def gk(table_hbm, idx_hbm, out_hbm, idx_v, rows_v, sem):
        wid = lax.axis_index("s") * nc + lax.axis_index("c")
        base = wid * b_per_w
        for ci in range(nchunks):
            off = base + ci * chunk
            pltpu.sync_copy(idx_hbm.at[pl.ds(off, chunk)], idx_v)
            pltpu.async_copy(table_hbm.at[idx_v], rows_v, sem).wait()
            pltpu.sync_copy(rows_v, out_hbm.at[pl.ds(off, chunk)])

    return gk(codebook, ids)


def kernel(z_e, codebook):
    b, t, c = z_e.shape
    k = codebook.shape[0]
    m = b * t
    z2d = z_e.reshape(m, c)

    nz_in = jnp.sum(jnp.square(z_e), axis=2).reshape(m, 1)
    ne_in = jnp.sum(jnp.square(codebook), axis=1).reshape(1, k)
    sim2d, ids3d, loss_parts = _main_call(z2d, codebook.T, nz_in, ne_in)
    ids = ids3d.reshape(m)
    zq2d = _gather_call(codebook, ids)

    loss_mean = jnp.sum(loss_parts) / m
    loss_vq = loss_mean + loss_mean * 0.25

    return (zq2d.reshape(b, t, c),
            sim2d.reshape(b, t, k),
            ids.reshape(b, t),
            loss_vq)


# speedup vs baseline: 1.2765x; 1.2765x over previous
import functools

import jax
import jax.numpy as jnp
from jax import lax
from jax.experimental import pallas as pl
from jax.experimental.pallas import tpu as pltpu
from jax.experimental.pallas import tpu_sc as plsc

TILE_M = 2048
TILE_N = 1024


def _main_body(z_ref, cb_ref, nz_ref, ne_ref, sim_ref, ids_ref, loss_ref,
               runmin_ref, runids_ref, cbt_ref):
    i = pl.program_id(0)
    j = pl.program_id(1)
    nj = pl.num_programs(1)
    z = z_ref[...]

    @pl.when(i == 0)
    def _():
        cbt_ref[:, pl.ds(j * TILE_N, TILE_N)] = cb_ref[...].T

    cbt = cbt_ref[:, pl.ds(j * TILE_N, TILE_N)]
    dot = lax.dot_general(z, cbt, (((1,), (0,)), ((), ())),
                          preferred_element_type=jnp.float32)
    nz = nz_ref[...]
    ne = ne_ref[...]
    sim_ref[...] = dot * lax.rsqrt(nz) * lax.rsqrt(ne)
    dist = (-2.0 * dot + nz) + ne
    lmin = jnp.min(dist, axis=1, keepdims=True)
    col = lax.broadcasted_iota(jnp.int32, dist.shape, 1)
    larg = jnp.min(jnp.where(dist == lmin, col, jnp.int32(2 ** 30)),
                   axis=1, keepdims=True) + j * TILE_N

    @pl.when(j == 0)
    def _():
        runmin_ref[...] = lmin
        runids_ref[...] = larg

    @pl.when(j > 0)
    def _():
        better = lmin < runmin_ref[...]
        runmin_ref[...] = jnp.where(better, lmin, runmin_ref[...])
        runids_ref[...] = jnp.where(better, larg, runids_ref[...])

    @pl.when(j == nj - 1)
    def _():
        ids_ref[...] = runids_ref[...].T.reshape(1, 1, TILE_M)
        loss_ref[0, 0, 0] = jnp.sum(jnp.sqrt(jnp.maximum(runmin_ref[...], 0.0)))


def _main_call(z2d, cb_in, nz_in, ne_in, interpret=False):
    m, c = z2d.shape
    k = cb_in.shape[0]
    gm = m // TILE_M
    grid = (gm, k // TILE_N)
    return pl.pallas_call(
        _main_body,
        grid=grid,
        in_specs=[
            pl.BlockSpec((TILE_M, c), lambda i, j: (i, 0)),
            pl.BlockSpec((TILE_N, c), lambda i, j: (jnp.where(i == 0, j, 0), 0)),
            pl.BlockSpec((TILE_M, 1), lambda i, j: (i, 0)),
            pl.BlockSpec((1, TILE_N), lambda i, j: (0, j)),
        ],
        out_specs=[
            pl.BlockSpec((TILE_M, TILE_N), lambda i, j: (i, j)),
            pl.BlockSpec((1, 1, TILE_M), lambda i, j: (i, 0, 0)),
            pl.BlockSpec((1, 1, 1), lambda i, j: (i, 0, 0),
                         memory_space=pltpu.SMEM),
        ],
        out_shape=[
            jax.ShapeDtypeStruct((m, k), jnp.float32),
            jax.ShapeDtypeStruct((gm, 1, TILE_M), jnp.int32),
            jax.ShapeDtypeStruct((gm, 1, 1), jnp.float32),
        ],
        scratch_shapes=[pltpu.VMEM((TILE_M, 1), jnp.float32),
                        pltpu.VMEM((TILE_M, 1), jnp.int32),
                        pltpu.VMEM((c, k), jnp.float32)],
        compiler_params=pltpu.CompilerParams(
            dimension_semantics=("parallel", "arbitrary")),
        interpret=interpret,
    )(z2d, cb_in, nz_in, ne_in)


def _gather_call(codebook, ids):
    info = plsc.get_sparse_core_info()
    nc, ns = info.num_cores, info.num_subcores
    nw = nc * ns
    m = ids.shape[0]
    d = codebook.shape[1]
    b_per_w = m // nw
    chunk = 128
    nchunks = b_per_w // chunk
    mesh = plsc.VectorSubcoreMesh(core_axis_name="c", subcore_axis_name="s")

    @functools.partial(
        pl.kernel,
        mesh=mesh,
        out_type=jax.ShapeDtypeStruct((m, d), jnp.float32),
        scratch_types=[
            pltpu.VMEM((chunk,), jnp.int32),
            pltpu.VMEM((chunk, d), jnp.float32),
            pltpu.SemaphoreType.DMA,
        ],
    )
    def gk(table_hbm, idx_hbm, out_hbm, idx_v, rows_v, sem):
        wid = lax.axis_index("s") * nc + lax.axis_index("c")
        base = wid * b_per_w
        for ci in range(nchunks):
            off = base + ci * chunk
            pltpu.sync_copy(idx_hbm.at[pl.ds(off, chunk)], idx_v)
            pltpu.async_copy(table_hbm.at[idx_v], rows_v, sem).wait()
            pltpu.sync_copy(rows_v, out_hbm.at[pl.ds(off, chunk)])

    return gk(codebook, ids)


def kernel(z_e, codebook):
    b, t, c = z_e.shape
    k = codebook.shape[0]
    m = b * t
    z2d = z_e.reshape(m, c)

    nz_in = jnp.sum(jnp.square(z_e), axis=2).reshape(m, 1)
    ne_in = jnp.sum(jnp.square(codebook), axis=1).reshape(1, k)
    sim2d, ids3d, loss_parts = _main_call(z2d, codebook, nz_in, ne_in)
    ids = ids3d.reshape(m)
    zq2d = _gather_call(codebook, ids)

    loss_mean = jnp.sum(loss_parts) / m
    loss_vq = loss_mean + loss_mean * 0.25

    return (zq2d.reshape(b, t, c),
            sim2d.reshape(b, t, k),
            ids.reshape(b, t),
            loss_vq)


# speedup vs baseline: 1.3436x; 1.0526x over previous
import functools

import jax
import jax.numpy as jnp
from jax import lax
from jax.experimental import pallas as pl
from jax.experimental.pallas import tpu as pltpu
from jax.experimental.pallas import tpu_sc as plsc

TILE_M = 2048
TILE_N = 1024


def _main_body(z_ref, cbt2_ref, nz_ref, ne_ref, sim_ref, ids_ref, loss_ref,
               runmin_ref, runids_ref):
    j = pl.program_id(1)
    nj = pl.num_programs(1)
    z = z_ref[...]
    cbt2 = cbt2_ref[...]
    dot2 = lax.dot_general(z, cbt2, (((1,), (0,)), ((), ())),
                           preferred_element_type=jnp.float32)
    nz = nz_ref[...]
    ne = ne_ref[...]
    sim_ref[...] = dot2 * (-0.5 * lax.rsqrt(nz)) * lax.rsqrt(ne)
    dist = (dot2 + nz) + ne
    lmin = jnp.min(dist, axis=1, keepdims=True)
    col = lax.broadcasted_iota(jnp.int32, dist.shape, 1)
    larg = jnp.min(jnp.where(dist == lmin, col, jnp.int32(2 ** 30)),
                   axis=1, keepdims=True) + j * TILE_N

    @pl.when(j == 0)
    def _():
        runmin_ref[...] = lmin
        runids_ref[...] = larg

    @pl.when(j > 0)
    def _():
        better = lmin < runmin_ref[...]
        runmin_ref[...] = jnp.where(better, lmin, runmin_ref[...])
        runids_ref[...] = jnp.where(better, larg, runids_ref[...])

    @pl.when(j == nj - 1)
    def _():
        ids_ref[...] = runids_ref[...].T.reshape(1, 1, TILE_M)
        loss_ref[0, 0, 0] = jnp.sum(jnp.sqrt(jnp.maximum(runmin_ref[...], 0.0)))


def _main_call(z2d, cbt2_in, nz_in, ne_in, interpret=False):
    m, c = z2d.shape
    k = cbt2_in.shape[1]
    gm = m // TILE_M
    grid = (gm, k // TILE_N)
    return pl.pallas_call(
        _main_body,
        grid=grid,
        in_specs=[
            pl.BlockSpec((TILE_M, c), lambda i, j: (i, 0)),
            pl.BlockSpec((c, TILE_N), lambda i, j: (0, j)),
            pl.BlockSpec((TILE_M, 1), lambda i, j: (i, 0)),
            pl.BlockSpec((1, TILE_N), lambda i, j: (0, j)),
        ],
        out_specs=[
            pl.BlockSpec((TILE_M, TILE_N), lambda i, j: (i, j)),
            pl.BlockSpec((1, 1, TILE_M), lambda i, j: (i, 0, 0)),
            pl.BlockSpec((1, 1, 1), lambda i, j: (i, 0, 0),
                         memory_space=pltpu.SMEM),
        ],
        out_shape=[
            jax.ShapeDtypeStruct((m, k), jnp.float32),
            jax.ShapeDtypeStruct((gm, 1, TILE_M), jnp.int32),
            jax.ShapeDtypeStruct((gm, 1, 1), jnp.float32),
        ],
        scratch_shapes=[pltpu.VMEM((TILE_M, 1), jnp.float32),
                        pltpu.VMEM((TILE_M, 1), jnp.int32)],
        compiler_params=pltpu.CompilerParams(
            dimension_semantics=("parallel", "arbitrary")),
        interpret=interpret,
    )(z2d, cbt2_in, nz_in, ne_in)


def _gather_call(codebook, ids):
    info = plsc.get_sparse_core_info()
    nc, ns = info.num_cores, info.num_subcores
    nw = nc * ns
    m = ids.shape[0]
    d = codebook.shape[1]
    b_per_w = m // nw
    chunk = 128
    nchunks = b_per_w // chunk
    mesh = plsc.VectorSubcoreMesh(core_axis_name="c", subcore_axis_name="s")

    @functools.partial(
        pl.kernel,
        mesh=mesh,
        out_type=jax.ShapeDtypeStruct((m, d), jnp.float32),
        scratch_types=[
            pltpu.VMEM((b_per_w,), jnp.int32),
            pltpu.VMEM((chunk, d), jnp.float32),
            pltpu.VMEM((chunk, d), jnp.float32),
            pltpu.SemaphoreType.DMA,
            pltpu.SemaphoreType.DMA,
        ],
    )
    def gk(table_hbm, idx_hbm, out_hbm, idx_v, rows0, rows1, sem0, sem1):
        wid = lax.axis_index("s") * nc + lax.axis_index("c")
        base = wid * b_per_w
        pltpu.sync_copy(idx_hbm.at[pl.ds(base, b_per_w)], idx_v)
        rows = (rows0, rows1)
        sems = (sem0, sem1)
        handles = [None, None]
        for ci in range(nchunks):
            handles[ci % 2] = pltpu.async_copy(
                table_hbm.at[idx_v.at[pl.ds(ci * chunk, chunk)]],
                rows[ci % 2], sems[ci % 2])
            if ci > 0:
                handles[(ci - 1) % 2].wait()
                pltpu.sync_copy(rows[(ci - 1) % 2],
                                out_hbm.at[pl.ds(base + (ci - 1) * chunk,
                                                 chunk)])
        handles[(nchunks - 1) % 2].wait()
        pltpu.sync_copy(rows[(nchunks - 1) % 2],
                        out_hbm.at[pl.ds(base + (nchunks - 1) * chunk, chunk)])

    return gk(codebook, ids)


def kernel(z_e, codebook):
    b, t, c = z_e.shape
    k = codebook.shape[0]
    m = b * t
    z2d = z_e.reshape(m, c)

    cbt2 = (codebook * -2.0).T
    nz_in = jnp.sum(jnp.square(z_e), axis=2).reshape(m, 1)
    ne_in = jnp.sum(jnp.square(codebook), axis=1).reshape(1, k)
    sim2d, ids3d, loss_parts = _main_call(z2d, cbt2, nz_in, ne_in)
    ids = ids3d.reshape(m)
    zq2d = _gather_call(codebook, ids)

    loss_mean = jnp.sum(loss_parts) / m
    loss_vq = loss_mean + loss_mean * 0.25

    return (zq2d.reshape(b, t, c),
            sim2d.reshape(b, t, k),
            ids.reshape(b, t),
            loss_vq)
